# baseline (speedup 1.0000x reference)
import functools

import jax
import jax.numpy as jnp
from jax import lax
from jax.experimental import pallas as pl
from jax.experimental.pallas import tpu as pltpu
from jax.experimental.pallas import tpu_sc as plsc

NC = 2
NS = 16
NW = NC * NS
CH = 128


def _sc_gather(idx3, table128, n_ch, per_w):
    mesh = plsc.VectorSubcoreMesh(
        core_axis_name="c", subcore_axis_name="s",
        num_cores=NC, num_subcores=NS)

    @functools.partial(
        pl.kernel,
        out_type=jax.ShapeDtypeStruct((NW * per_w, 128), jnp.float32),
        mesh=mesh,
        scratch_types=[
            pltpu.VMEM((n_ch, CH), jnp.int32),
            pltpu.VMEM((2, CH, 128), jnp.float32),
            pltpu.SemaphoreType.DMA,
            pltpu.SemaphoreType.DMA,
        ],
    )
    def body(idx_hbm, table_hbm, out_hbm, idx_v, rows_v, sem_g, sem_w):
        wid = lax.axis_index("s") * NC + lax.axis_index("c")
        base = wid * per_w
        pltpu.sync_copy(idx_hbm.at[wid], idx_v)
        writes = [None, None]
        for j in range(n_ch):
            s = j % 2
            if writes[s] is not None:
                writes[s].wait()
            pltpu.async_copy(
                table_hbm.at[idx_v.at[j]], rows_v.at[s], sem_g
            ).wait()
            writes[s] = pltpu.async_copy(
                rows_v.at[s], out_hbm.at[pl.ds(base + j * CH, CH)], sem_w
            )
        for w in writes:
            if w is not None:
                w.wait()

    return body(idx3, table128)


def _mlp_body(ga_ref, gb_ref, pa_ref, pb_ref, wa_ref, wb_ref, b1_ref,
              w2_ref, b2_ref, o_ref):
    bb = ga_ref.shape[1]
    ge64 = lax.broadcasted_iota(jnp.int32, (bb, 128), 1) >= 64
    pa = pa_ref[0]
    pb = pb_ref[0]
    ma = jnp.where(ge64, pa, 1.0 - pa)
    mb = jnp.where(ge64, pb, 1.0 - pb)
    am = ga_ref[0] * ma
    bm = gb_ref[0] * mb
    h = jnp.dot(am, wa_ref[...], preferred_element_type=jnp.float32)
    h = h + jnp.dot(bm, wb_ref[...], preferred_element_type=jnp.float32)
    h = jnp.maximum(h + b1_ref[...], 0.0)
    o_ref[...] = (
        jnp.dot(h, w2_ref[...], preferred_element_type=jnp.float32)
        + b2_ref[...]
    )


def kernel(x, emb, W1, b1, W2, b2):
    B = x.shape[0]
    H = emb.shape[1]
    L = W2.shape[0]

    total = 2 * B
    per_w = total // NW
    n_ch = per_w // CH

    xt = x.T
    idx3 = (xt >> 1).reshape(NW, n_ch, CH)
    parf = (xt & 1).astype(jnp.float32).reshape(2, B, 1)

    table128 = emb.reshape(-1, 2 * H)
    g = _sc_gather(idx3, table128, n_ch, per_w)
    g3 = g.reshape(2, B, 2 * H)

    Wa = W1[:, :H].T
    Wb = W1[:, H:].T
    WaS = jnp.concatenate([Wa, Wa], axis=0)
    WbS = jnp.concatenate([Wb, Wb], axis=0)
    W2T = W2.T

    BB = 2048
    grid = (B // BB,)
    out = pl.pallas_call(
        _mlp_body,
        grid=grid,
        in_specs=[
            pl.BlockSpec((1, BB, 2 * H), lambda i: (0, i, 0)),
            pl.BlockSpec((1, BB, 2 * H), lambda i: (1, i, 0)),
            pl.BlockSpec((1, BB, 1), lambda i: (0, i, 0)),
            pl.BlockSpec((1, BB, 1), lambda i: (1, i, 0)),
            pl.BlockSpec((2 * H, H), lambda i: (0, 0)),
            pl.BlockSpec((2 * H, H), lambda i: (0, 0)),
            pl.BlockSpec((1, H), lambda i: (0, 0)),
            pl.BlockSpec((H, L), lambda i: (0, 0)),
            pl.BlockSpec((1, L), lambda i: (0, 0)),
        ],
        out_specs=pl.BlockSpec((BB, L), lambda i: (i, 0)),
        out_shape=jax.ShapeDtypeStruct((B, L), jnp.float32),
    )(g3, g3, parf, parf, WaS, WbS, b1.reshape(1, H), W2T, b2.reshape(1, L))
    return out


# speedup vs baseline: 1.6149x; 1.6149x over previous
import functools

import jax
import jax.numpy as jnp
from jax import lax
from jax.experimental import pallas as pl
from jax.experimental.pallas import tpu as pltpu
from jax.experimental.pallas import tpu_sc as plsc

NC = 2
NS = 16
NW = NC * NS
CH = 128
OFF = 499968
CBLK = 1792


def _repack_body(ta_ref, tb_ref, eye_ref, out_ref):
    dn = (((0,), (0,)), ((), ()))
    qa = lax.dot_general(ta_ref[...], eye_ref[...], dn,
                         preferred_element_type=jnp.float32)
    qb = lax.dot_general(tb_ref[...], eye_ref[...], dn,
                         preferred_element_type=jnp.float32)
    out_ref[...] = jnp.concatenate([qa, qb], axis=1)


def _repack(embT, n_rows):
    H = embT.shape[0]
    nblk = (n_rows + CBLK - 1) // CBLK
    off_blk = OFF // CBLK
    eye = jnp.eye(H, dtype=jnp.float32)
    return pl.pallas_call(
        _repack_body,
        grid=(nblk,),
        in_specs=[
            pl.BlockSpec((H, CBLK), lambda i: (0, i)),
            pl.BlockSpec((H, CBLK), lambda i: (0, i + off_blk)),
            pl.BlockSpec((H, H), lambda i: (0, 0)),
        ],
        out_specs=pl.BlockSpec((CBLK, 2 * H), lambda i: (i, 0)),
        out_shape=jax.ShapeDtypeStruct((n_rows, 2 * H), jnp.float32),
    )(embT, embT, eye)


def _sc_gather(idx3, table, n_ch, per_w):
    mesh = plsc.VectorSubcoreMesh(
        core_axis_name="c", subcore_axis_name="s",
        num_cores=NC, num_subcores=NS)

    @functools.partial(
        pl.kernel,
        out_type=jax.ShapeDtypeStruct((NW * per_w, 128), jnp.float32),
        mesh=mesh,
        scratch_types=[
            pltpu.VMEM((n_ch, CH), jnp.int32),
            pltpu.VMEM((2, CH, 128), jnp.float32),
            pltpu.SemaphoreType.DMA,
            pltpu.SemaphoreType.DMA,
        ],
    )
    def body(idx_hbm, table_hbm, out_hbm, idx_v, rows_v, sem_g, sem_w):
        wid = lax.axis_index("s") * NC + lax.axis_index("c")
        base = wid * per_w
        pltpu.sync_copy(idx_hbm.at[wid], idx_v)
        writes = [None, None]
        for j in range(n_ch):
            s = j % 2
            if writes[s] is not None:
                writes[s].wait()
            pltpu.async_copy(
                table_hbm.at[idx_v.at[j]], rows_v.at[s], sem_g
            ).wait()
            writes[s] = pltpu.async_copy(
                rows_v.at[s], out_hbm.at[pl.ds(base + j * CH, CH)], sem_w
            )
        for w in writes:
            if w is not None:
                w.wait()

    return body(idx3, table)


def _mlp_body(ga_ref, gb_ref, pa_ref, pb_ref, wa_ref, wb_ref, b1_ref,
              w2_ref, b2_ref, o_ref):
    bb = ga_ref.shape[1]
    ge64 = lax.broadcasted_iota(jnp.int32, (bb, 128), 1) >= 64
    pa = pa_ref[0]
    pb = pb_ref[0]
    ma = jnp.where(ge64, pa, 1.0 - pa)
    mb = jnp.where(ge64, pb, 1.0 - pb)
    am = ga_ref[0] * ma
    bm = gb_ref[0] * mb
    h = jnp.dot(am, wa_ref[...], preferred_element_type=jnp.float32)
    h = h + jnp.dot(bm, wb_ref[...], preferred_element_type=jnp.float32)
    h = jnp.maximum(h + b1_ref[...], 0.0)
    o_ref[...] = (
        jnp.dot(h, w2_ref[...], preferred_element_type=jnp.float32)
        + b2_ref[...]
    )


def kernel(x, emb, W1, b1, W2, b2):
    B = x.shape[0]
    H = emb.shape[1]
    L = W2.shape[0]
    n_rows = OFF + 64

    total = 2 * B
    per_w = total // NW
    n_ch = per_w // CH

    xt = x.T
    hi = (xt >= n_rows).astype(jnp.int32)
    idx3 = (xt - OFF * hi).reshape(NW, n_ch, CH)
    parf = hi.astype(jnp.float32).reshape(2, B, 1)

    embT = emb.T
    P = _repack(embT, n_rows)
    g = _sc_gather(idx3, P, n_ch, per_w)
    g3 = g.reshape(2, B, 2 * H)

    Wa = W1[:, :H].T
    Wb = W1[:, H:].T
    WaS = jnp.concatenate([Wa, Wa], axis=0)
    WbS = jnp.concatenate([Wb, Wb], axis=0)
    W2T = W2.T

    BB = 2048
    grid = (B // BB,)
    out = pl.pallas_call(
        _mlp_body,
        grid=grid,
        in_specs=[
            pl.BlockSpec((1, BB, 2 * H), lambda i: (0, i, 0)),
            pl.BlockSpec((1, BB, 2 * H), lambda i: (1, i, 0)),
            pl.BlockSpec((1, BB, 1), lambda i: (0, i, 0)),
            pl.BlockSpec((1, BB, 1), lambda i: (1, i, 0)),
            pl.BlockSpec((2 * H, H), lambda i: (0, 0)),
            pl.BlockSpec((2 * H, H), lambda i: (0, 0)),
            pl.BlockSpec((1, H), lambda i: (0, 0)),
            pl.BlockSpec((H, L), lambda i: (0, 0)),
            pl.BlockSpec((1, L), lambda i: (0, 0)),
        ],
        out_specs=pl.BlockSpec((BB, L), lambda i: (i, 0)),
        out_shape=jax.ShapeDtypeStruct((B, L), jnp.float32),
    )(g3, g3, parf, parf, WaS, WbS, b1.reshape(1, H), W2T, b2.reshape(1, L))
    return out


# speedup vs baseline: 2.2923x; 1.4195x over previous
import functools

import jax
import jax.numpy as jnp
from jax import lax
from jax.experimental import pallas as pl
from jax.experimental.pallas import tpu as pltpu
from jax.experimental.pallas import tpu_sc as plsc

NC = 2
NS = 16
NW = NC * NS
CH = 128
OFF = 499968
CBLK = 8064


def _repack_body(ta_ref, tb_ref, eye_ref, out_ref):
    dn = (((0,), (0,)), ((), ()))
    qa = lax.dot_general(ta_ref[...], eye_ref[...], dn,
                         preferred_element_type=jnp.float32)
    qb = lax.dot_general(tb_ref[...], eye_ref[...], dn,
                         preferred_element_type=jnp.float32)
    out_ref[...] = jnp.concatenate([qa, qb], axis=1)


def _repack(embT, n_rows):
    H = embT.shape[0]
    nblk = (n_rows + CBLK - 1) // CBLK
    off_blk = OFF // CBLK
    eye = jnp.eye(H, dtype=jnp.float32)
    return pl.pallas_call(
        _repack_body,
        grid=(nblk,),
        in_specs=[
            pl.BlockSpec((H, CBLK), lambda i: (0, i)),
            pl.BlockSpec((H, CBLK), lambda i: (0, i + off_blk)),
            pl.BlockSpec((H, H), lambda i: (0, 0)),
        ],
        out_specs=pl.BlockSpec((CBLK, 2 * H), lambda i: (i, 0)),
        out_shape=jax.ShapeDtypeStruct((n_rows, 2 * H), jnp.float32),
    )(embT, embT, eye)


def _sc_gather(idx3, table, n_ch, per_w):
    mesh = plsc.VectorSubcoreMesh(
        core_axis_name="c", subcore_axis_name="s",
        num_cores=NC, num_subcores=NS)

    @functools.partial(
        pl.kernel,
        out_type=jax.ShapeDtypeStruct((NW * per_w, 128), jnp.float32),
        mesh=mesh,
        scratch_types=[
            pltpu.VMEM((n_ch, CH), jnp.int32),
            pltpu.VMEM((2, CH, 128), jnp.float32),
            pltpu.SemaphoreType.DMA,
            pltpu.SemaphoreType.DMA,
        ],
    )
    def body(idx_hbm, table_hbm, out_hbm, idx_v, rows_v, sem_g, sem_w):
        wid = lax.axis_index("s") * NC + lax.axis_index("c")
        base = wid * per_w
        pltpu.sync_copy(idx_hbm.at[wid], idx_v)
        writes = [None, None]
        for j in range(n_ch):
            s = j % 2
            if writes[s] is not None:
                writes[s].wait()
            pltpu.async_copy(
                table_hbm.at[idx_v.at[j]], rows_v.at[s], sem_g
            ).wait()
            writes[s] = pltpu.async_copy(
                rows_v.at[s], out_hbm.at[pl.ds(base + j * CH, CH)], sem_w
            )
        for w in writes:
            if w is not None:
                w.wait()

    return body(idx3, table)


def _mlp_body(ga_ref, gb_ref, pa_ref, pb_ref, wa_ref, wb_ref, b1_ref,
              w2_ref, b2_ref, o_ref):
    bb = ga_ref.shape[1]
    ge64 = lax.broadcasted_iota(jnp.int32, (bb, 128), 1) >= 64
    pa = pa_ref[0]
    pb = pb_ref[0]
    ma = jnp.where(ge64, pa, 1.0 - pa)
    mb = jnp.where(ge64, pb, 1.0 - pb)
    am = ga_ref[0] * ma
    bm = gb_ref[0] * mb
    h = jnp.dot(am, wa_ref[...], preferred_element_type=jnp.float32)
    h = h + jnp.dot(bm, wb_ref[...], preferred_element_type=jnp.float32)
    h = jnp.maximum(h + b1_ref[...], 0.0)
    o_ref[...] = (
        jnp.dot(h, w2_ref[...], preferred_element_type=jnp.float32)
        + b2_ref[...]
    )


def kernel(x, emb, W1, b1, W2, b2):
    B = x.shape[0]
    H = emb.shape[1]
    L = W2.shape[0]
    n_rows = OFF + 64

    total = 2 * B
    per_w = total // NW
    n_ch = per_w // CH

    xt = x.T
    hi = (xt >= n_rows).astype(jnp.int32)
    idx3 = (xt - OFF * hi).reshape(NW, n_ch, CH)
    parf = hi.astype(jnp.float32).reshape(2, B, 1)

    embT = emb.T
    P = _repack(embT, n_rows)
    g = _sc_gather(idx3, P, n_ch, per_w)
    g3 = g.reshape(2, B, 2 * H)

    Wa = W1[:, :H].T
    Wb = W1[:, H:].T
    WaS = jnp.concatenate([Wa, Wa], axis=0)
    WbS = jnp.concatenate([Wb, Wb], axis=0)
    W2T = W2.T

    BB = 2048
    grid = (B // BB,)
    out = pl.pallas_call(
        _mlp_body,
        grid=grid,
        in_specs=[
            pl.BlockSpec((1, BB, 2 * H), lambda i: (0, i, 0)),
            pl.BlockSpec((1, BB, 2 * H), lambda i: (1, i, 0)),
            pl.BlockSpec((1, BB, 1), lambda i: (0, i, 0)),
            pl.BlockSpec((1, BB, 1), lambda i: (1, i, 0)),
            pl.BlockSpec((2 * H, H), lambda i: (0, 0)),
            pl.BlockSpec((2 * H, H), lambda i: (0, 0)),
            pl.BlockSpec((1, H), lambda i: (0, 0)),
            pl.BlockSpec((H, L), lambda i: (0, 0)),
            pl.BlockSpec((1, L), lambda i: (0, 0)),
        ],
        out_specs=pl.BlockSpec((BB, L), lambda i: (i, 0)),
        out_shape=jax.ShapeDtypeStruct((B, L), jnp.float32),
    )(g3, g3, parf, parf, WaS, WbS, b1.reshape(1, H), W2T, b2.reshape(1, L))
    return out
